# Initial kernel scaffold; baseline (speedup 1.0000x reference)
import jax
import jax.numpy as jnp
from jax.experimental import pallas as pl


def kernel(x, edge_index, W1l, b1, W1r, W2l, b2, W2r, W3l, b3, W3r):
    raise NotImplementedError("write your pallas kernel here")


# speedup vs baseline: 5.2128x; 5.2128x over previous
import functools

import jax
import jax.numpy as jnp
from jax import lax
from jax.experimental import pallas as pl
from jax.experimental.pallas import tpu as pltpu
from jax.experimental.pallas import tpu_sc as plsc

N = 10000
E = 320000
D = 128

NC, NS = 2, 16
NW = NC * NS
EPW = E // NW
CH = 128
NFULL = EPW // CH
TAIL = EPW - NFULL * CH
NP = 10240
RPT = NP // NS


def _make_sc_agg(W):
    mesh = plsc.VectorSubcoreMesh(core_axis_name="c", subcore_axis_name="s")

    @functools.partial(
        pl.kernel,
        mesh=mesh,
        out_type=jax.ShapeDtypeStruct((NC, NP, W), jnp.float32),
        scratch_types=[
            pltpu.VMEM((CH,), jnp.int32),
            pltpu.VMEM((CH,), jnp.int32),
            pltpu.VMEM((CH, W), jnp.float32),
            pltpu.VMEM((TAIL,), jnp.int32),
            pltpu.VMEM((TAIL,), jnp.int32),
            pltpu.VMEM((TAIL, W), jnp.float32),
            pltpu.VMEM_SHARED((NP, W), jnp.float32),
            pltpu.SemaphoreType.DMA,
        ],
    )
    def agg(h_hbm, src_hbm, dst_hbm, z_hbm, out_hbm,
            sidx, didx, rows, sidx_t, didx_t, rows_t, acc, sem):
        c = lax.axis_index("c")
        s = lax.axis_index("s")
        wid = s * NC + c
        base = wid * EPW
        r0 = s * RPT

        pltpu.sync_copy(z_hbm.at[pl.ds(r0, RPT)], acc.at[pl.ds(r0, RPT)])
        plsc.subcore_barrier()

        def body(i, carry):
            off = base + i * CH
            pltpu.sync_copy(src_hbm.at[pl.ds(off, CH)], sidx)
            pltpu.sync_copy(dst_hbm.at[pl.ds(off, CH)], didx)
            pltpu.async_copy(h_hbm.at[sidx], rows, sem).wait()
            pltpu.sync_copy(rows, acc.at[didx], add=True)
            return carry

        lax.fori_loop(0, NFULL, body, 0)

        offt = base + NFULL * CH
        pltpu.sync_copy(src_hbm.at[pl.ds(offt, TAIL)], sidx_t)
        pltpu.sync_copy(dst_hbm.at[pl.ds(offt, TAIL)], didx_t)
        pltpu.async_copy(h_hbm.at[sidx_t], rows_t, sem).wait()
        pltpu.sync_copy(rows_t, acc.at[didx_t], add=True)

        plsc.subcore_barrier()
        pltpu.sync_copy(acc.at[pl.ds(r0, RPT)], out_hbm.at[c, pl.ds(r0, RPT)])

    return agg


_AGG = _make_sc_agg(D)

_CNT_MESH = plsc.VectorSubcoreMesh(core_axis_name="c", subcore_axis_name="s")


@functools.partial(
    pl.kernel,
    mesh=_CNT_MESH,
    out_type=jax.ShapeDtypeStruct((NC, NP, D), jnp.float32),
    scratch_types=[
        pltpu.VMEM((CH,), jnp.int32),
        pltpu.VMEM((TAIL,), jnp.int32),
        pltpu.VMEM((CH, D), jnp.float32),
        pltpu.VMEM_SHARED((NP, D), jnp.float32),
    ],
)
def _sc_counts(dst_hbm, ones_hbm, z_hbm, out_hbm, didx, didx_t, ones_v, acc):
    c = lax.axis_index("c")
    s = lax.axis_index("s")
    wid = s * NC + c
    base = wid * EPW
    r0 = s * RPT

    pltpu.sync_copy(ones_hbm, ones_v)
    pltpu.sync_copy(z_hbm.at[pl.ds(r0, RPT)], acc.at[pl.ds(r0, RPT)])
    plsc.subcore_barrier()

    def body(i, carry):
        off = base + i * CH
        pltpu.sync_copy(dst_hbm.at[pl.ds(off, CH)], didx)
        pltpu.sync_copy(ones_v, acc.at[didx], add=True)
        return carry

    lax.fori_loop(0, NFULL, body, 0)

    offt = base + NFULL * CH
    pltpu.sync_copy(dst_hbm.at[pl.ds(offt, TAIL)], didx_t)
    pltpu.sync_copy(ones_v.at[pl.ds(0, TAIL)], acc.at[didx_t], add=True)

    plsc.subcore_barrier()
    pltpu.sync_copy(acc.at[pl.ds(r0, RPT)], out_hbm.at[c, pl.ds(r0, RPT)])

BN = 1000


def _tc1_body(p0_ref, p1_ref, c0_ref, c1_ref, x_ref, wl_ref, b_ref, wr_ref,
              h_ref, rc_ref):
    cnt = c0_ref[:, :1] + c1_ref[:, :1]
    rc = 1.0 / jnp.maximum(cnt, 1.0)
    mean = (p0_ref[...] + p1_ref[...]) * rc
    acc = jnp.dot(mean, wl_ref[...], preferred_element_type=jnp.float32)
    acc = acc + jnp.dot(x_ref[...], wr_ref[...], preferred_element_type=jnp.float32)
    acc = acc + b_ref[...]
    h_ref[...] = jnp.maximum(acc, 0.0)
    rc_ref[...] = rc


def _tc_layer1(p0, p1, c0, c1, x, Wl, b, Wr):
    return pl.pallas_call(
        _tc1_body,
        grid=(N // BN,),
        in_specs=[
            pl.BlockSpec((BN, D), lambda i: (i, 0)),
            pl.BlockSpec((BN, D), lambda i: (i, 0)),
            pl.BlockSpec((BN, D), lambda i: (i, 0)),
            pl.BlockSpec((BN, D), lambda i: (i, 0)),
            pl.BlockSpec((BN, D), lambda i: (i, 0)),
            pl.BlockSpec((D, D), lambda i: (0, 0)),
            pl.BlockSpec((1, D), lambda i: (0, 0)),
            pl.BlockSpec((D, D), lambda i: (0, 0)),
        ],
        out_specs=[
            pl.BlockSpec((BN, D), lambda i: (i, 0)),
            pl.BlockSpec((BN, 1), lambda i: (i, 0)),
        ],
        out_shape=[
            jax.ShapeDtypeStruct((N, D), jnp.float32),
            jax.ShapeDtypeStruct((N, 1), jnp.float32),
        ],
    )(p0, p1, c0, c1, x, Wl, b, Wr)


def _make_tc23_body(relu):
    def body(p0_ref, p1_ref, x_ref, rc_ref, wl_ref, b_ref, wr_ref, h_ref):
        mean = (p0_ref[...] + p1_ref[...]) * rc_ref[...]
        acc = jnp.dot(mean, wl_ref[...], preferred_element_type=jnp.float32)
        acc = acc + jnp.dot(x_ref[...], wr_ref[...], preferred_element_type=jnp.float32)
        acc = acc + b_ref[...]
        h_ref[...] = jnp.maximum(acc, 0.0) if relu else acc
    return body


def _tc_layer23(p0, p1, x, rc, Wl, b, Wr, relu):
    return pl.pallas_call(
        _make_tc23_body(relu),
        grid=(N // BN,),
        in_specs=[
            pl.BlockSpec((BN, D), lambda i: (i, 0)),
            pl.BlockSpec((BN, D), lambda i: (i, 0)),
            pl.BlockSpec((BN, D), lambda i: (i, 0)),
            pl.BlockSpec((BN, 1), lambda i: (i, 0)),
            pl.BlockSpec((D, D), lambda i: (0, 0)),
            pl.BlockSpec((1, D), lambda i: (0, 0)),
            pl.BlockSpec((D, D), lambda i: (0, 0)),
        ],
        out_specs=pl.BlockSpec((BN, D), lambda i: (i, 0)),
        out_shape=jax.ShapeDtypeStruct((N, D), jnp.float32),
    )(p0, p1, x, rc, Wl, b, Wr)


def kernel(x, edge_index, W1l, b1, W1r, W2l, b2, W2r, W3l, b3, W3r):
    src = edge_index[0].astype(jnp.int32)
    dst = edge_index[1].astype(jnp.int32)

    z = jnp.zeros((NP, D), jnp.float32)
    ones_blk = jnp.ones((CH, D), jnp.float32)

    cp = _sc_counts(dst, ones_blk, z)
    p = _AGG(x, src, dst, z)
    h1, rc = _tc_layer1(p[0, :N], p[1, :N], cp[0, :N], cp[1, :N], x,
                        W1l, b1.reshape(1, D), W1r)

    p = _AGG(h1, src, dst, z)
    h2 = _tc_layer23(p[0, :N], p[1, :N], h1, rc, W2l, b2.reshape(1, D), W2r, relu=True)

    p = _AGG(h2, src, dst, z)
    h3 = _tc_layer23(p[0, :N], p[1, :N], h2, rc, W3l, b3.reshape(1, D), W3r, relu=False)
    return h3


# speedup vs baseline: 10.9799x; 2.1064x over previous
import functools

import jax
import jax.numpy as jnp
from jax import lax
from jax.experimental import pallas as pl
from jax.experimental.pallas import tpu as pltpu
from jax.experimental.pallas import tpu_sc as plsc

N = 10000
E = 320000
D = 128

NC, NS = 2, 16
NW = NC * NS
EPW = E // NW
CH = 128
EPWP = 10240
NCH = EPWP // CH
PAD = EPWP - EPW
NP = 10240
RPT = NP // NS

_MESH = plsc.VectorSubcoreMesh(core_axis_name="c", subcore_axis_name="s")


@functools.partial(
    pl.kernel,
    mesh=_MESH,
    out_type=jax.ShapeDtypeStruct((NC, NP, D), jnp.float32),
    scratch_types=[
        pltpu.VMEM((EPWP,), jnp.int32),
        pltpu.VMEM((CH,), jnp.int32),
        pltpu.VMEM((CH,), jnp.int32),
        pltpu.VMEM((CH, D), jnp.float32),
        pltpu.VMEM((CH, D), jnp.float32),
        pltpu.VMEM_SHARED((NP, D), jnp.float32),
        pltpu.SemaphoreType.DMA,
        pltpu.SemaphoreType.DMA,
        pltpu.SemaphoreType.DMA,
        pltpu.SemaphoreType.DMA,
    ],
)
def _sc_agg(h_hbm, src_hbm, dst_hbm, z_hbm, out_hbm,
            sidx, dA, dB, rowsA, rowsB, acc, gsA, gsB, dsA, dsB):
    c = lax.axis_index("c")
    s = lax.axis_index("s")
    wid = s * NC + c
    base = wid * EPWP
    r0 = s * RPT

    pltpu.sync_copy(src_hbm.at[pl.ds(base, EPWP)], sidx)
    pltpu.sync_copy(z_hbm.at[pl.ds(r0, RPT)], acc.at[pl.ds(r0, RPT)])
    plsc.subcore_barrier()

    def fire(ci, dbuf, rbuf, dsem, gsem):
        pltpu.async_copy(dst_hbm.at[pl.ds(base + ci * CH, CH)], dbuf, dsem)
        pltpu.async_copy(h_hbm.at[sidx.at[pl.ds(ci * CH, CH)]], rbuf, gsem)

    def drain_scatter(dbuf, rbuf, dsem, gsem):
        pltpu.make_async_copy(dst_hbm.at[pl.ds(0, CH)], dbuf, dsem).wait()
        pltpu.make_async_copy(h_hbm.at[pl.ds(0, CH)], rbuf, gsem).wait()
        pltpu.sync_copy(rbuf, acc.at[dbuf], add=True)

    fire(0, dA, rowsA, dsA, gsA)

    def body(j, carry):
        c0 = 2 * j
        fire(c0 + 1, dB, rowsB, dsB, gsB)
        drain_scatter(dA, rowsA, dsA, gsA)

        @pl.when(j < NCH // 2 - 1)
        def _():
            fire(c0 + 2, dA, rowsA, dsA, gsA)

        drain_scatter(dB, rowsB, dsB, gsB)
        return carry

    lax.fori_loop(0, NCH // 2, body, 0)

    plsc.subcore_barrier()
    pltpu.sync_copy(acc.at[pl.ds(r0, RPT)], out_hbm.at[c, pl.ds(r0, RPT)])


@functools.partial(
    pl.kernel,
    mesh=_MESH,
    out_type=jax.ShapeDtypeStruct((NC, NP), jnp.float32),
    scratch_types=[
        pltpu.VMEM((CH,), jnp.int32),
        pltpu.VMEM((CH,), jnp.int32),
        pltpu.VMEM((CH,), jnp.float32),
        pltpu.VMEM_SHARED((NP,), jnp.float32),
        pltpu.SemaphoreType.DMA,
        pltpu.SemaphoreType.DMA,
    ],
)
def _sc_counts(dst_hbm, z_hbm, out_hbm, dA, dB, ones_v, acc, dsA, dsB):
    c = lax.axis_index("c")
    s = lax.axis_index("s")
    wid = s * NC + c
    base = wid * EPWP
    r0 = s * RPT

    for k in range(CH // 16):
        ones_v[pl.ds(16 * k, 16)] = jnp.full((16,), 1.0, jnp.float32)
    pltpu.sync_copy(z_hbm.at[pl.ds(r0, RPT)], acc.at[pl.ds(r0, RPT)])
    plsc.subcore_barrier()

    def fire(ci, dbuf, dsem):
        pltpu.async_copy(dst_hbm.at[pl.ds(base + ci * CH, CH)], dbuf, dsem)

    def drain_scatter(dbuf, dsem):
        pltpu.make_async_copy(dst_hbm.at[pl.ds(0, CH)], dbuf, dsem).wait()
        pltpu.sync_copy(ones_v, acc.at[dbuf], add=True)

    fire(0, dA, dsA)

    def body(j, carry):
        c0 = 2 * j
        fire(c0 + 1, dB, dsB)
        drain_scatter(dA, dsA)

        @pl.when(j < NCH // 2 - 1)
        def _():
            fire(c0 + 2, dA, dsA)

        drain_scatter(dB, dsB)
        return carry

    lax.fori_loop(0, NCH // 2, body, 0)

    plsc.subcore_barrier()
    pltpu.sync_copy(acc.at[pl.ds(r0, RPT)], out_hbm.at[c, pl.ds(r0, RPT)])


BN = 1000


def _tc1_body(p0_ref, p1_ref, c0_ref, c1_ref, x_ref, wl_ref, b_ref, wr_ref,
              h_ref, rc_ref):
    cnt = c0_ref[...] + c1_ref[...]
    rc = 1.0 / jnp.maximum(cnt, 1.0)
    mean = (p0_ref[...] + p1_ref[...]) * rc
    acc = jnp.dot(mean, wl_ref[...], preferred_element_type=jnp.float32)
    acc = acc + jnp.dot(x_ref[...], wr_ref[...], preferred_element_type=jnp.float32)
    acc = acc + b_ref[...]
    h_ref[...] = jnp.maximum(acc, 0.0)
    rc_ref[...] = rc


def _tc_layer1(p0, p1, c0, c1, x, Wl, b, Wr):
    return pl.pallas_call(
        _tc1_body,
        grid=(N // BN,),
        in_specs=[
            pl.BlockSpec((BN, D), lambda i: (i, 0)),
            pl.BlockSpec((BN, D), lambda i: (i, 0)),
            pl.BlockSpec((BN, 1), lambda i: (i, 0)),
            pl.BlockSpec((BN, 1), lambda i: (i, 0)),
            pl.BlockSpec((BN, D), lambda i: (i, 0)),
            pl.BlockSpec((D, D), lambda i: (0, 0)),
            pl.BlockSpec((1, D), lambda i: (0, 0)),
            pl.BlockSpec((D, D), lambda i: (0, 0)),
        ],
        out_specs=[
            pl.BlockSpec((BN, D), lambda i: (i, 0)),
            pl.BlockSpec((BN, 1), lambda i: (i, 0)),
        ],
        out_shape=[
            jax.ShapeDtypeStruct((N, D), jnp.float32),
            jax.ShapeDtypeStruct((N, 1), jnp.float32),
        ],
    )(p0, p1, c0, c1, x, Wl, b, Wr)


def _make_tc23_body(relu):
    def body(p0_ref, p1_ref, x_ref, rc_ref, wl_ref, b_ref, wr_ref, h_ref):
        mean = (p0_ref[...] + p1_ref[...]) * rc_ref[...]
        acc = jnp.dot(mean, wl_ref[...], preferred_element_type=jnp.float32)
        acc = acc + jnp.dot(x_ref[...], wr_ref[...], preferred_element_type=jnp.float32)
        acc = acc + b_ref[...]
        h_ref[...] = jnp.maximum(acc, 0.0) if relu else acc
    return body


def _tc_layer23(p0, p1, x, rc, Wl, b, Wr, relu):
    return pl.pallas_call(
        _make_tc23_body(relu),
        grid=(N // BN,),
        in_specs=[
            pl.BlockSpec((BN, D), lambda i: (i, 0)),
            pl.BlockSpec((BN, D), lambda i: (i, 0)),
            pl.BlockSpec((BN, D), lambda i: (i, 0)),
            pl.BlockSpec((BN, 1), lambda i: (i, 0)),
            pl.BlockSpec((D, D), lambda i: (0, 0)),
            pl.BlockSpec((1, D), lambda i: (0, 0)),
            pl.BlockSpec((D, D), lambda i: (0, 0)),
        ],
        out_specs=pl.BlockSpec((BN, D), lambda i: (i, 0)),
        out_shape=jax.ShapeDtypeStruct((N, D), jnp.float32),
    )(p0, p1, x, rc, Wl, b, Wr)


def _pad_edges(src, dst):
    srcw = src.reshape(NW, EPW)
    dstw = dst.reshape(NW, EPW)
    pad_ids = jnp.arange(NW * PAD, dtype=jnp.int32).reshape(NW, PAD)
    src_pad = pad_ids % N
    dst_pad = N + pad_ids % (NP - N)
    src_p = jnp.concatenate([srcw, src_pad], axis=1).reshape(-1)
    dst_p = jnp.concatenate([dstw, dst_pad], axis=1).reshape(-1)
    return src_p, dst_p


def kernel(x, edge_index, W1l, b1, W1r, W2l, b2, W2r, W3l, b3, W3r):
    src = edge_index[0].astype(jnp.int32)
    dst = edge_index[1].astype(jnp.int32)
    src_p, dst_p = _pad_edges(src, dst)

    z = jnp.zeros((NP, D), jnp.float32)
    z1 = jnp.zeros((NP,), jnp.float32)

    cp = _sc_counts(dst_p, z1)
    p = _sc_agg(x, src_p, dst_p, z)
    h1, rc = _tc_layer1(p[0, :N], p[1, :N],
                        cp[0, :N].reshape(N, 1), cp[1, :N].reshape(N, 1), x,
                        W1l, b1.reshape(1, D), W1r)

    p = _sc_agg(h1, src_p, dst_p, z)
    h2 = _tc_layer23(p[0, :N], p[1, :N], h1, rc, W2l, b2.reshape(1, D), W2r,
                     relu=True)

    p = _sc_agg(h2, src_p, dst_p, z)
    h3 = _tc_layer23(p[0, :N], p[1, :N], h2, rc, W3l, b3.reshape(1, D), W3r,
                     relu=False)
    return h3
